# Initial kernel scaffold; baseline (speedup 1.0000x reference)
import jax
import jax.numpy as jnp
from jax.experimental import pallas as pl


def kernel(x, edge_index, edge_attr, batch, u, params):
    raise NotImplementedError("write your pallas kernel here")


# speedup vs baseline: 1.1356x; 1.1356x over previous
import functools

import jax
import jax.numpy as jnp
from jax import lax
from jax.experimental import pallas as pl
from jax.experimental.pallas import tpu as pltpu
from jax.experimental.pallas import tpu_sc as plsc

N = 10240
E = 20480
B = 256
HID = 32
EPS = 1e-5

NC = 2
NS = 16
NW = NC * NS
CHUNK = 128
NCHUNK = E // (NW * CHUNK)

BLK = 512


def _sc_gather_body(x_hbm, src_hbm, out_hbm, idx_v, rows_v, sem):
    wid = lax.axis_index("s") * NC + lax.axis_index("c")
    pltpu.sync_copy(src_hbm.at[wid], idx_v)
    descs = [
        pltpu.async_copy(x_hbm.at[idx_v.at[j]], rows_v.at[j], sem)
        for j in range(NCHUNK)
    ]
    for d in descs:
        d.wait()
    pltpu.sync_copy(rows_v, out_hbm.at[wid])


def _sc_gather(x, src3, in_ch):
    mesh = plsc.VectorSubcoreMesh(core_axis_name="c", subcore_axis_name="s")
    fn = pl.kernel(
        _sc_gather_body,
        out_type=jax.ShapeDtypeStruct((NW, NCHUNK, CHUNK, in_ch), jnp.float32),
        mesh=mesh,
        scratch_types=[
            pltpu.VMEM((NCHUNK, CHUNK), jnp.int32),
            pltpu.VMEM((NCHUNK, CHUNK, in_ch), jnp.float32),
            pltpu.SemaphoreType.DMA,
        ],
        compiler_params=pltpu.CompilerParams(use_tc_tiling_on_sc=False),
    )
    return fn(x, src3)


def _sc_scatter_body(msg_hbm, dst_hbm, zeros_hbm, out_hbm,
                     idx_v, msg_v, acc_shared):
    cid = lax.axis_index("c")
    sid = lax.axis_index("s")
    wid = sid * NC + cid

    @pl.when(sid == 0)
    def _():
        pltpu.sync_copy(zeros_hbm, acc_shared)

    plsc.subcore_barrier()
    pltpu.sync_copy(dst_hbm.at[wid], idx_v)
    pltpu.sync_copy(msg_hbm.at[wid], msg_v)
    for j in range(NCHUNK):
        pltpu.sync_copy(msg_v.at[j], acc_shared.at[idx_v.at[j]], add=True)
    plsc.subcore_barrier()

    @pl.when(sid == 0)
    def _():
        pltpu.sync_copy(acc_shared, out_hbm.at[cid])


def _sc_scatter(msg4, dst3, zeros_n):
    mesh = plsc.VectorSubcoreMesh(core_axis_name="c", subcore_axis_name="s")
    fn = pl.kernel(
        _sc_scatter_body,
        out_type=jax.ShapeDtypeStruct((NC, N, HID), jnp.float32),
        mesh=mesh,
        scratch_types=[
            pltpu.VMEM((NCHUNK, CHUNK), jnp.int32),
            pltpu.VMEM((NCHUNK, CHUNK, HID), jnp.float32),
            pltpu.VMEM_SHARED((N, HID), jnp.float32),
        ],
        compiler_params=pltpu.CompilerParams(use_tc_tiling_on_sc=False),
    )
    return fn(msg4, dst3, zeros_n)


def _tc_edge_body(ea_ref, xs_ref, w1_ref, b1_ref, w2_ref, b2_ref, out_ref):
    in_ch = xs_ref.shape[1]
    h = jnp.maximum(ea_ref[...] @ w1_ref[...] + b1_ref[...], 0.0)
    w = h @ w2_ref[...] + b2_ref[...]
    xs = xs_ref[...]
    xe = jnp.reshape(
        jnp.broadcast_to(xs[:, :, None], (BLK, in_ch, HID)),
        (BLK, in_ch * HID))
    prod = w * xe
    while prod.shape[1] > HID:
        half = prod.shape[1] // 2
        prod = prod[:, :half] + prod[:, half:]
    out_ref[...] = prod


def _tc_edge(edge_attr, xs, p, in_ch):
    w1 = p['W1']
    b1 = p['b1'].reshape(1, 128)
    w2 = p['W2']
    b2 = p['b2'].reshape(1, in_ch * HID)
    grid = (E // BLK,)
    return pl.pallas_call(
        _tc_edge_body,
        grid=grid,
        in_specs=[
            pl.BlockSpec((BLK, 4), lambda i: (i, 0)),
            pl.BlockSpec((BLK, in_ch), lambda i: (i, 0)),
            pl.BlockSpec((4, 128), lambda i: (0, 0)),
            pl.BlockSpec((1, 128), lambda i: (0, 0)),
            pl.BlockSpec((128, in_ch * HID), lambda i: (0, 0)),
            pl.BlockSpec((1, in_ch * HID), lambda i: (0, 0)),
        ],
        out_specs=pl.BlockSpec((BLK, HID), lambda i: (i, 0)),
        out_shape=jax.ShapeDtypeStruct((E, HID), jnp.float32),
    )(edge_attr, xs, w1, b1, w2, b2)


def _tc_node_body(agg_ref, x_ref, root_ref, bias_ref, gamma_ref, beta_ref,
                  out_ref):
    y = (agg_ref[0] + agg_ref[1]
         + x_ref[...] @ root_ref[...] + bias_ref[...])
    mean = jnp.mean(y, axis=0, keepdims=True)
    d = y - mean
    var = jnp.mean(d * d, axis=0, keepdims=True)
    xn = d * lax.rsqrt(var + EPS)
    out_ref[...] = jnp.maximum(xn * gamma_ref[...] + beta_ref[...], 0.0)


def _tc_node(agg2, x, p):
    return pl.pallas_call(
        _tc_node_body,
        out_shape=jax.ShapeDtypeStruct((N, HID), jnp.float32),
    )(agg2, x, p['root'], p['bias'].reshape(1, HID),
      p['gamma'].reshape(1, HID), p['beta'].reshape(1, HID))


def _tc_final_body(x_ref, bcol_ref, brow_ref, u_ref, w1_ref, b1_ref,
                   w2_ref, b2_ref, out_ref):
    scan = x_ref[...]
    bc = bcol_ref[...]
    k = 1
    while k < 2 * N:
        rs = jnp.concatenate([scan[N - k:], scan[:N - k]], axis=0)
        rb = jnp.concatenate([bc[N - k:], bc[:N - k]], axis=0)
        scan = jnp.where(bc == rb, jnp.maximum(scan, rs), scan)
        k *= 2
    br = brow_ref[...]
    nxt = jnp.concatenate(
        [br[:, 1:], jnp.full((1, 1), -1, jnp.int32)], axis=1)
    is_end = br != nxt
    pooled = jnp.zeros((B, HID), jnp.float32)
    exists = jnp.zeros((B, 1), jnp.float32)
    CH = 2048
    for c0 in range(0, N, CH):
        ids = lax.broadcasted_iota(jnp.int32, (B, CH), 0)
        oh = jnp.where(
            (ids == br[:, c0:c0 + CH]) & is_end[:, c0:c0 + CH], 1.0, 0.0)
        pooled = pooled + oh @ scan[c0:c0 + CH]
        exists = exists + jnp.sum(oh, axis=1, keepdims=True)
    pooled = jnp.where(exists > 0.5, pooled, -jnp.inf)
    hcat = jnp.concatenate([pooled, u_ref[...]], axis=1)
    hmlp = jnp.maximum(hcat @ w1_ref[...] + b1_ref[...], 0.0)
    out_ref[...] = hmlp @ w2_ref[...] + b2_ref[...]


def _tc_final(x, batch_col, batch_row, u, pp):
    return pl.pallas_call(
        _tc_final_body,
        out_shape=jax.ShapeDtypeStruct((B, 1), jnp.float32),
    )(x, batch_col, batch_row, u, pp['W1'], pp['b1'].reshape(1, 64),
      pp['W2'], pp['b2'].reshape(1, 1))


def kernel(x, edge_index, edge_attr, batch, u, params):
    src3 = edge_index[0].reshape(NW, NCHUNK, CHUNK)
    dst3 = edge_index[1].reshape(NW, NCHUNK, CHUNK)
    zeros_n = jnp.zeros((N, HID), jnp.float32)
    batch_col = batch.reshape(N, 1)
    batch_row = batch.reshape(1, N)
    for l in range(4):
        p = params['layer%d' % l]
        in_ch = x.shape[1]
        xs4 = _sc_gather(x, src3, in_ch)
        msg = _tc_edge(edge_attr, xs4.reshape(E, in_ch), p, in_ch)
        agg2 = _sc_scatter(msg.reshape(NW, NCHUNK, CHUNK, HID), dst3, zeros_n)
        x = _tc_node(agg2, x, p)
    out = _tc_final(x, batch_col, batch_row, u, params['post'])
    return out.reshape(-1)


# speedup vs baseline: 2.9046x; 2.5577x over previous
import jax
import jax.numpy as jnp
from jax import lax
from jax.experimental import pallas as pl
from jax.experimental.pallas import tpu as pltpu
from jax.experimental.pallas import tpu_sc as plsc

N = 10240
E = 20480
B = 256
HID = 32
EPS = 1e-5
W = 128

NC = 2
NS = 16
NW = NC * NS
CHUNK = 128
NCHUNK = E // (NW * CHUNK)
EPW = E // NW

HALF = N // 2
NDUMP = 128
ACC_ROWS = HALF + NDUMP
SCHUNK = E // (NS * CHUNK)
SEPW = E // NS
NBUF = 4

BLK = 1024


def _sc_gather_body(x_hbm, src_hbm, out_hbm, idx_v, rows_v, sem):
    wid = lax.axis_index("s") * NC + lax.axis_index("c")
    pltpu.sync_copy(src_hbm.at[wid], idx_v)
    descs = [
        pltpu.async_copy(x_hbm.at[idx_v.at[j]], rows_v.at[j], sem)
        for j in range(NCHUNK)
    ]
    for d in descs:
        d.wait()
    for j in range(NCHUNK):
        pltpu.sync_copy(
            rows_v.at[j],
            out_hbm.at[pl.ds(pl.multiple_of(wid * EPW + j * CHUNK, 8),
                             CHUNK)])


def _sc_gather(x_pad, src3):
    mesh = plsc.VectorSubcoreMesh(core_axis_name="c", subcore_axis_name="s")
    fn = pl.kernel(
        _sc_gather_body,
        out_type=jax.ShapeDtypeStruct((E, W), jnp.float32),
        mesh=mesh,
        scratch_types=[
            pltpu.VMEM((NCHUNK, CHUNK), jnp.int32),
            pltpu.VMEM((NCHUNK, CHUNK, W), jnp.float32),
            pltpu.SemaphoreType.DMA,
        ],
    )
    return fn(x_pad, src3)


def _sc_scatter_body(msg_hbm, dst_hbm, zeros_hbm, out_hbm,
                     idx_v, idxm_v, msg_v, ldsem, acc_shared):
    cid = lax.axis_index("c")
    sid = lax.axis_index("s")
    zrows = ACC_ROWS // NS
    zoff = pl.multiple_of(sid * zrows, 8)
    pltpu.sync_copy(zeros_hbm.at[pl.ds(zoff, zrows)],
                    acc_shared.at[pl.ds(zoff, zrows)])
    pltpu.sync_copy(dst_hbm.at[sid], idx_v)
    base = cid * HALF
    lane = lax.iota(jnp.int32, 16)
    for j in range(SCHUNK):
        for t in range(CHUNK // 16):
            v = idx_v[j, pl.ds(t * 16, 16)]
            rel = v - base
            ok = (rel >= 0) & (rel < HALF)
            dump = (HALF + ((j * (CHUNK // 16) + t) % (NDUMP // 16)) * 16
                    + lane)
            idxm_v[j, pl.ds(t * 16, 16)] = jnp.where(ok, rel, dump)
    plsc.subcore_barrier()
    ld = [None] * SCHUNK
    for j in range(NBUF):
        ld[j] = pltpu.async_copy(
            msg_hbm.at[pl.ds(pl.multiple_of(sid * SEPW + j * CHUNK, 8),
                             CHUNK)],
            msg_v.at[j % NBUF], ldsem)
    for j in range(SCHUNK):
        ld[j].wait()
        pltpu.sync_copy(msg_v.at[j % NBUF], acc_shared.at[idxm_v.at[j]],
                        add=True)
        nxt = j + NBUF
        if nxt < SCHUNK:
            ld[nxt] = pltpu.async_copy(
                msg_hbm.at[pl.ds(pl.multiple_of(sid * SEPW + nxt * CHUNK, 8),
                                 CHUNK)],
                msg_v.at[nxt % NBUF], ldsem)
    plsc.subcore_barrier()
    orows = HALF // NS
    pltpu.sync_copy(
        acc_shared.at[pl.ds(pl.multiple_of(sid * orows, 8), orows)],
        out_hbm.at[pl.ds(pl.multiple_of(cid * HALF + sid * orows, 8),
                         orows)])


def _sc_scatter(msg, dst3, zeros_acc):
    mesh = plsc.VectorSubcoreMesh(core_axis_name="c", subcore_axis_name="s")
    fn = pl.kernel(
        _sc_scatter_body,
        out_type=jax.ShapeDtypeStruct((N, W), jnp.float32),
        mesh=mesh,
        scratch_types=[
            pltpu.VMEM((SCHUNK, CHUNK), jnp.int32),
            pltpu.VMEM((SCHUNK, CHUNK), jnp.int32),
            pltpu.VMEM((NBUF, CHUNK, W), jnp.float32),
            pltpu.SemaphoreType.DMA,
            pltpu.VMEM_SHARED((ACC_ROWS, W), jnp.float32),
        ],
    )
    return fn(msg, dst3, zeros_acc)


def _tc_edge_body(ea_ref, xs_ref, w1_ref, b1_ref, w2_ref, b2_ref, r_ref,
                  out_ref):
    in_ch = r_ref.shape[0]
    h = jnp.maximum(ea_ref[...] @ w1_ref[...] + b1_ref[...], 0.0)
    w = h @ w2_ref[...] + b2_ref[...]
    xe = xs_ref[:, :in_ch] @ r_ref[...]
    p = w * xe
    while p.shape[1] > W:
        half = p.shape[1] // 2
        p = p[:, :half] + p[:, half:]
    out_ref[...] = p


def _tc_edge(edge_attr, xs, p, in_ch, r_mat):
    w1 = p['W1']
    b1 = p['b1'].reshape(1, 128)
    w2 = p['W2']
    b2 = p['b2'].reshape(1, in_ch * HID)
    grid = (E // BLK,)
    return pl.pallas_call(
        _tc_edge_body,
        grid=grid,
        in_specs=[
            pl.BlockSpec((BLK, 4), lambda i: (i, 0)),
            pl.BlockSpec((BLK, W), lambda i: (i, 0)),
            pl.BlockSpec((4, 128), lambda i: (0, 0)),
            pl.BlockSpec((1, 128), lambda i: (0, 0)),
            pl.BlockSpec((128, in_ch * HID), lambda i: (0, 0)),
            pl.BlockSpec((1, in_ch * HID), lambda i: (0, 0)),
            pl.BlockSpec((in_ch, in_ch * HID), lambda i: (0, 0)),
        ],
        out_specs=pl.BlockSpec((BLK, W), lambda i: (i, 0)),
        out_shape=jax.ShapeDtypeStruct((E, W), jnp.float32),
    )(edge_attr, xs, w1, b1, w2, b2, r_mat)


def _fold_and_norm(agg128, x_pad, in_ch, root, bias, gamma, beta):
    agg64 = agg128[:, :64] + agg128[:, 64:]
    agg = agg64[:, :HID] + agg64[:, HID:]
    y = agg + x_pad[:, :in_ch] @ root + bias
    mean = jnp.mean(y, axis=0, keepdims=True)
    d = y - mean
    var = jnp.mean(d * d, axis=0, keepdims=True)
    xn = d * lax.rsqrt(var + EPS)
    return jnp.maximum(xn * gamma + beta, 0.0)


def _tc_node_body(agg_ref, x_ref, root_ref, bias_ref, gamma_ref, beta_ref,
                  out_ref):
    in_ch = root_ref.shape[0]
    xn = _fold_and_norm(agg_ref[...], x_ref[...], in_ch, root_ref[...],
                        bias_ref[...], gamma_ref[...], beta_ref[...])
    out_ref[...] = jnp.concatenate(
        [xn, jnp.zeros((N, W - HID), jnp.float32)], axis=1)


def _tc_node(agg2, x_pad, p, in_ch):
    return pl.pallas_call(
        _tc_node_body,
        out_shape=jax.ShapeDtypeStruct((N, W), jnp.float32),
    )(agg2, x_pad, p['root'], p['bias'].reshape(1, HID),
      p['gamma'].reshape(1, HID), p['beta'].reshape(1, HID))


def _tc_final_body(agg_ref, x_ref, root_ref, bias_ref, gamma_ref, beta_ref,
                   bcol_ref, brow_ref, u_ref, w1_ref, b1_ref, w2_ref, b2_ref,
                   out_ref):
    in_ch = root_ref.shape[0]
    scan = _fold_and_norm(agg_ref[...], x_ref[...], in_ch, root_ref[...],
                          bias_ref[...], gamma_ref[...], beta_ref[...])
    bc = bcol_ref[...]
    k = 1
    while k < 2 * N:
        rs = jnp.concatenate([scan[N - k:], scan[:N - k]], axis=0)
        rb = jnp.concatenate([bc[N - k:], bc[:N - k]], axis=0)
        scan = jnp.where(bc == rb, jnp.maximum(scan, rs), scan)
        k *= 2
    br = brow_ref[...]
    nxt = jnp.concatenate(
        [br[:, 1:], jnp.full((1, 1), -1, jnp.int32)], axis=1)
    is_end = br != nxt
    pooled = jnp.zeros((B, HID), jnp.float32)
    exists = jnp.zeros((B, 1), jnp.float32)
    CH = 2048
    for c0 in range(0, N, CH):
        ids = lax.broadcasted_iota(jnp.int32, (B, CH), 0)
        oh = jnp.where(
            (ids == br[:, c0:c0 + CH]) & is_end[:, c0:c0 + CH], 1.0, 0.0)
        pooled = pooled + oh @ scan[c0:c0 + CH]
        exists = exists + jnp.sum(oh, axis=1, keepdims=True)
    pooled = jnp.where(exists > 0.5, pooled, -jnp.inf)
    hcat = jnp.concatenate([pooled, u_ref[...]], axis=1)
    hmlp = jnp.maximum(hcat @ w1_ref[...] + b1_ref[...], 0.0)
    out_ref[...] = hmlp @ w2_ref[...] + b2_ref[...]


def _tc_final(agg2, x_pad, p, in_ch, batch_col, batch_row, u, pp):
    return pl.pallas_call(
        _tc_final_body,
        out_shape=jax.ShapeDtypeStruct((B, 1), jnp.float32),
    )(agg2, x_pad, p['root'], p['bias'].reshape(1, HID),
      p['gamma'].reshape(1, HID), p['beta'].reshape(1, HID),
      batch_col, batch_row, u, pp['W1'], pp['b1'].reshape(1, 64),
      pp['W2'], pp['b2'].reshape(1, 1))


def kernel(x, edge_index, edge_attr, batch, u, params):
    src3 = edge_index[0].reshape(NW, NCHUNK, CHUNK)
    dst3 = edge_index[1].reshape(NS, SCHUNK, CHUNK)
    zeros_acc = jnp.zeros((ACC_ROWS, W), jnp.float32)
    batch_col = jnp.broadcast_to(batch.reshape(N, 1), (N, HID))
    batch_row = batch.reshape(1, N)
    x_pad = jnp.pad(x, ((0, 0), (0, W - x.shape[1])))
    r_mats = {}
    out = None
    for l in range(4):
        p = params['layer%d' % l]
        in_ch = 16 if l == 0 else HID
        if in_ch not in r_mats:
            r_mats[in_ch] = jnp.kron(
                jnp.eye(in_ch, dtype=jnp.float32),
                jnp.ones((1, HID), jnp.float32))
        xs = _sc_gather(x_pad, src3)
        msg = _tc_edge(edge_attr, xs, p, in_ch, r_mats[in_ch])
        agg = _sc_scatter(msg, dst3, zeros_acc)
        if l < 3:
            x_pad = _tc_node(agg, x_pad, p, in_ch)
        else:
            out = _tc_final(agg, x_pad, p, in_ch, batch_col, batch_row, u,
                            params['post'])
    return out.reshape(-1)
